# baseline (speedup 1.0000x reference)
import functools

import jax
import jax.numpy as jnp
from jax import lax
from jax.experimental import pallas as pl
from jax.experimental.pallas import tpu as pltpu
from jax.experimental.pallas import tpu_sc as plsc

NW = 32
T = 256
IB = 128
VL = 16


def _sc_embed(seq_blk, e1_blk, e2_blk, we, wpe, n, dw, dp, npos):
    d = dw + 2 * dp
    per_w = n // NW
    steps = per_w // T
    k = T // IB

    mesh = plsc.VectorSubcoreMesh(core_axis_name="c", subcore_axis_name="s")

    @functools.partial(
        pl.kernel,
        out_type=jax.ShapeDtypeStruct((n, d), jnp.float32),
        mesh=mesh,
        compiler_params=pltpu.CompilerParams(use_tc_tiling_on_sc=False),
        scratch_types=[
            pltpu.VMEM((k, IB), jnp.int32),
            pltpu.VMEM((k, IB), jnp.int32),
            pltpu.VMEM((T,), jnp.int32),
            pltpu.VMEM((T,), jnp.int32),
            pltpu.VMEM((T,), jnp.int32),
            pltpu.VMEM((T,), jnp.int32),
            pltpu.VMEM((T, dw), jnp.float32),
            pltpu.VMEM((T, dw), jnp.float32),
            pltpu.VMEM((T, d), jnp.float32),
            pltpu.VMEM((T, d), jnp.float32),
            pltpu.VMEM((npos, dp), jnp.float32),
            pltpu.SemaphoreType.DMA,
            pltpu.SemaphoreType.DMA,
            pltpu.SemaphoreType.DMA,
            pltpu.SemaphoreType.DMA,
        ],
    )
    def body(seq_hbm, e1_hbm, e2_hbm, we_hbm, wpe_hbm, out_hbm,
             si0, si1, p1a, p1b, p2a, p2b, w0, w1, rows0, rows1,
             wpe_v, gs0, gs1, ws0, ws1):
        cid = lax.axis_index("c")
        sid = lax.axis_index("s")
        wid = sid * 2 + cid
        sblk0 = wid * steps
        si_v = (si0, si1)
        p1_v = (p1a, p1b)
        p2_v = (p2a, p2b)
        w_v = (w0, w1)
        rows_v = (rows0, rows1)
        gsem = (gs0, gs1)
        wsem = (ws0, ws1)

        pltpu.sync_copy(wpe_hbm, wpe_v)

        def out_copy(b, row0):
            return pltpu.make_async_copy(
                rows_v[b], out_hbm.at[pl.ds(row0, T)], wsem[b])

        def gather_copies(b):
            return [
                pltpu.make_async_copy(
                    we_hbm.at[si_v[b].at[j]],
                    w_v[b].at[pl.ds(j * IB, IB)], gsem[b])
                for j in range(k)
            ]

        def load_and_gather(b, t):
            pltpu.sync_copy(seq_hbm.at[sblk0 + t], si_v[b])
            pltpu.sync_copy(e1_hbm.at[sblk0 + t], p1_v[b])
            pltpu.sync_copy(e2_hbm.at[sblk0 + t], p2_v[b])
            for c in gather_copies(b):
                c.start()

        def fuse_pos(b):
            p1r = p1_v[b]
            p2r = p2_v[b]
            rows = rows_v[b]

            def fuse(g, carry2):
                p1vec = p1r[pl.ds(g * VL, VL)]
                p2vec = p2r[pl.ds(g * VL, VL)]
                for u in range(VL):
                    row = g * VL + u
                    p1 = p1vec[u]
                    p2 = p2vec[u]
                    for c in range(dp // VL):
                        rows[row, pl.ds(dw + c * VL, VL)] = (
                            wpe_v[p1, pl.ds(c * VL, VL)])
                    for c in range(dp // VL):
                        rows[row, pl.ds(dw + dp + c * VL, VL)] = (
                            wpe_v[p2, pl.ds(c * VL, VL)])
                return carry2

            lax.fori_loop(0, T // VL, fuse, 0)

        def fuse_words(b):
            wr = w_v[b]
            rows = rows_v[b]

            def fuse(r, carry2):
                for u in range(2):
                    row = r * 2 + u
                    for c in range(dw // VL):
                        rows[row, pl.ds(c * VL, VL)] = (
                            wr[row, pl.ds(c * VL, VL)])
                return carry2

            lax.fori_loop(0, T // 2, fuse, 0)

        def step2(t2, carry):
            for b in range(2):
                t = t2 * 2 + b
                row0 = wid * per_w + t * T

                @pl.when(t2 >= 1)
                def _drain_prev():
                    out_copy(b, row0 - 2 * T).wait()

                load_and_gather(b, t)
                fuse_pos(b)
                for c in gather_copies(b):
                    c.wait()
                fuse_words(b)
                out_copy(b, row0).start()
            return carry

        lax.fori_loop(0, steps // 2, step2, 0)

        last0 = wid * per_w + (steps - 2) * T
        out_copy(0, last0).wait()
        out_copy(1, last0 + T).wait()

    return body(seq_blk, e1_blk, e2_blk, we, wpe)


def kernel(seq_inputs, e1_pos_inputs, e2_pos_inputs, we, wpe):
    b, l = seq_inputs.shape
    dw = we.shape[1]
    dp = wpe.shape[1]
    npos = wpe.shape[0]
    n = b * l
    assert n % (NW * T * 2) == 0 and T % IB == 0
    k = T // IB
    seq_blk = seq_inputs.reshape(n // T, k, IB)
    e1_blk = e1_pos_inputs.reshape(n // T, T)
    e2_blk = e2_pos_inputs.reshape(n // T, T)
    out = _sc_embed(seq_blk, e1_blk, e2_blk, we, wpe, n, dw, dp, npos)
    return out.reshape(b, l, dw + 2 * dp)


# speedup vs baseline: 1.2876x; 1.2876x over previous
import functools

import jax
import jax.numpy as jnp
from jax import lax
from jax.experimental import pallas as pl
from jax.experimental.pallas import tpu as pltpu
from jax.experimental.pallas import tpu_sc as plsc

NW = 32
T = 256
IB = 128
VL = 16


def _sc_embed(seq_blk, e1_blk, e2_blk, we, wpe, n, dw, dp, npos):
    d = dw + 2 * dp
    per_w = n // NW
    steps = per_w // T
    k = T // IB

    mesh = plsc.VectorSubcoreMesh(core_axis_name="c", subcore_axis_name="s")

    @functools.partial(
        pl.kernel,
        out_type=jax.ShapeDtypeStruct((n, d), jnp.float32),
        mesh=mesh,
        compiler_params=pltpu.CompilerParams(use_tc_tiling_on_sc=False),
        scratch_types=[
            pltpu.VMEM((k, IB), jnp.int32),
            pltpu.VMEM((k, IB), jnp.int32),
            pltpu.VMEM((T,), jnp.int32),
            pltpu.VMEM((T,), jnp.int32),
            pltpu.VMEM((T,), jnp.int32),
            pltpu.VMEM((T,), jnp.int32),
            pltpu.VMEM((T, dw), jnp.float32),
            pltpu.VMEM((T, dw), jnp.float32),
            pltpu.VMEM((T, 2 * dp), jnp.float32),
            pltpu.VMEM((T, 2 * dp), jnp.float32),
            pltpu.VMEM((npos, dp), jnp.float32),
            pltpu.SemaphoreType.DMA,
            pltpu.SemaphoreType.DMA,
            pltpu.SemaphoreType.DMA,
            pltpu.SemaphoreType.DMA,
        ],
    )
    def body(seq_hbm, e1_hbm, e2_hbm, we_hbm, wpe_hbm, out_hbm,
             si0, si1, p1a, p1b, p2a, p2b, w0, w1, rows0, rows1,
             wpe_v, gs0, gs1, ws0, ws1):
        cid = lax.axis_index("c")
        sid = lax.axis_index("s")
        wid = sid * 2 + cid
        sblk0 = wid * steps
        si_v = (si0, si1)
        p1_v = (p1a, p1b)
        p2_v = (p2a, p2b)
        w_v = (w0, w1)
        rows_v = (rows0, rows1)
        gsem = (gs0, gs1)
        wsem = (ws0, ws1)

        pltpu.sync_copy(wpe_hbm, wpe_v)

        def out_copies(b, row0):
            rows = pl.ds(row0, T)
            return [
                pltpu.make_async_copy(
                    w_v[b], out_hbm.at[rows, pl.ds(0, dw)], wsem[b]),
                pltpu.make_async_copy(
                    rows_v[b], out_hbm.at[rows, pl.ds(dw, 2 * dp)],
                    wsem[b]),
            ]

        def gather_copies(b):
            return [
                pltpu.make_async_copy(
                    we_hbm.at[si_v[b].at[j]],
                    w_v[b].at[pl.ds(j * IB, IB)], gsem[b])
                for j in range(k)
            ]

        def load_and_gather(b, t):
            pltpu.sync_copy(seq_hbm.at[sblk0 + t], si_v[b])
            pltpu.sync_copy(e1_hbm.at[sblk0 + t], p1_v[b])
            pltpu.sync_copy(e2_hbm.at[sblk0 + t], p2_v[b])
            for c in gather_copies(b):
                c.start()

        def fuse_pos(b):
            p1r = p1_v[b]
            p2r = p2_v[b]
            rows = rows_v[b]

            def fuse(g, carry2):
                p1vec = p1r[pl.ds(g * VL, VL)]
                p2vec = p2r[pl.ds(g * VL, VL)]
                for u in range(VL):
                    row = g * VL + u
                    p1 = p1vec[u]
                    p2 = p2vec[u]
                    for c in range(dp // VL):
                        rows[row, pl.ds(c * VL, VL)] = (
                            wpe_v[p1, pl.ds(c * VL, VL)])
                    for c in range(dp // VL):
                        rows[row, pl.ds(dp + c * VL, VL)] = (
                            wpe_v[p2, pl.ds(c * VL, VL)])
                return carry2

            lax.fori_loop(0, T // VL, fuse, 0)

        def step2(t2, carry):
            for b in range(2):
                t = t2 * 2 + b
                row0 = wid * per_w + t * T

                @pl.when(t2 >= 1)
                def _drain_prev():
                    for c in out_copies(b, row0 - 2 * T):
                        c.wait()

                load_and_gather(b, t)
                fuse_pos(b)
                for c in gather_copies(b):
                    c.wait()
                for c in out_copies(b, row0):
                    c.start()
            return carry

        lax.fori_loop(0, steps // 2, step2, 0)

        last0 = wid * per_w + (steps - 2) * T
        for c in out_copies(0, last0):
            c.wait()
        for c in out_copies(1, last0 + T):
            c.wait()

    return body(seq_blk, e1_blk, e2_blk, we, wpe)


def kernel(seq_inputs, e1_pos_inputs, e2_pos_inputs, we, wpe):
    b, l = seq_inputs.shape
    dw = we.shape[1]
    dp = wpe.shape[1]
    npos = wpe.shape[0]
    n = b * l
    assert n % (NW * T * 2) == 0 and T % IB == 0
    k = T // IB
    seq_blk = seq_inputs.reshape(n // T, k, IB)
    e1_blk = e1_pos_inputs.reshape(n // T, T)
    e2_blk = e2_pos_inputs.reshape(n // T, T)
    out = _sc_embed(seq_blk, e1_blk, e2_blk, we, wpe, n, dw, dp, npos)
    return out.reshape(b, l, dw + 2 * dp)


# speedup vs baseline: 1.2898x; 1.0017x over previous
import functools

import jax
import jax.numpy as jnp
from jax import lax
from jax.experimental import pallas as pl
from jax.experimental.pallas import tpu as pltpu
from jax.experimental.pallas import tpu_sc as plsc

NW = 32
T = 256
IB = 128
VL = 16


def _sc_embed(seq_blk, e1_blk, e2_blk, we, wpe, n, dw, dp, npos):
    d = dw + 2 * dp
    per_w = n // NW
    steps = per_w // T
    k = T // IB

    mesh = plsc.VectorSubcoreMesh(core_axis_name="c", subcore_axis_name="s")

    @functools.partial(
        pl.kernel,
        out_type=jax.ShapeDtypeStruct((n, d), jnp.float32),
        mesh=mesh,
        compiler_params=pltpu.CompilerParams(use_tc_tiling_on_sc=False),
        scratch_types=(
            [pltpu.VMEM((k, IB), jnp.int32)] * 3
            + [pltpu.VMEM((T,), jnp.int32)] * 6
            + [pltpu.VMEM((T, dw), jnp.float32)] * 3
            + [pltpu.VMEM((T, 2 * dp), jnp.float32)] * 3
            + [pltpu.VMEM((npos, dp), jnp.float32)]
            + [pltpu.SemaphoreType.DMA] * 6
        ),
    )
    def body(seq_hbm, e1_hbm, e2_hbm, we_hbm, wpe_hbm, out_hbm,
             si0, si1, si2, p1a, p1b, p1c, p2a, p2b, p2c,
             w0, w1, w2, rows0, rows1, rows2,
             wpe_v, gs0, gs1, gs2, ws0, ws1, ws2):
        cid = lax.axis_index("c")
        sid = lax.axis_index("s")
        wid = sid * 2 + cid
        sblk0 = wid * steps
        si_v = (si0, si1, si2)
        p1_v = (p1a, p1b, p1c)
        p2_v = (p2a, p2b, p2c)
        w_v = (w0, w1, w2)
        rows_v = (rows0, rows1, rows2)
        gsem = (gs0, gs1, gs2)
        wsem = (ws0, ws1, ws2)

        pltpu.sync_copy(wpe_hbm, wpe_v)

        def out_copies(b, row0):
            rows = pl.ds(row0, T)
            return [
                pltpu.make_async_copy(
                    w_v[b], out_hbm.at[rows, pl.ds(0, dw)], wsem[b]),
                pltpu.make_async_copy(
                    rows_v[b], out_hbm.at[rows, pl.ds(dw, 2 * dp)],
                    wsem[b]),
            ]

        def gather_copies(b):
            return [
                pltpu.make_async_copy(
                    we_hbm.at[si_v[b].at[j]],
                    w_v[b].at[pl.ds(j * IB, IB)], gsem[b])
                for j in range(k)
            ]

        def load_and_gather(b, t):
            pltpu.sync_copy(seq_hbm.at[sblk0 + t], si_v[b])
            pltpu.sync_copy(e1_hbm.at[sblk0 + t], p1_v[b])
            pltpu.sync_copy(e2_hbm.at[sblk0 + t], p2_v[b])
            for c in gather_copies(b):
                c.start()

        def fuse_pos(b):
            p1r = p1_v[b]
            p2r = p2_v[b]
            rows = rows_v[b]

            def fuse(g, carry2):
                p1vec = p1r[pl.ds(g * VL, VL)]
                p2vec = p2r[pl.ds(g * VL, VL)]
                for u in range(VL):
                    row = g * VL + u
                    p1 = p1vec[u]
                    p2 = p2vec[u]
                    for c in range(dp // VL):
                        rows[row, pl.ds(c * VL, VL)] = (
                            wpe_v[p1, pl.ds(c * VL, VL)])
                    for c in range(dp // VL):
                        rows[row, pl.ds(dp + c * VL, VL)] = (
                            wpe_v[p2, pl.ds(c * VL, VL)])
                return carry2

            lax.fori_loop(0, T // VL, fuse, 0)

        load_and_gather(0, 0)

        def step3(t3, carry):
            for u in range(3):
                t = t3 * 3 + u
                nb = (u + 1) % 3
                row0 = wid * per_w + t * T

                @pl.when(t >= 2)
                def _drain_old():
                    for c in out_copies(nb, row0 - 2 * T):
                        c.wait()

                load_and_gather(nb, t + 1)
                fuse_pos(u)
                for c in gather_copies(u):
                    c.wait()
                for c in out_copies(u, row0):
                    c.start()
            return carry

        lax.fori_loop(0, (steps - 1) // 3, step3, 0)

        tl = steps - 1
        trow0 = wid * per_w + tl * T
        for c in out_copies(1, trow0 - 2 * T):
            c.wait()
        fuse_pos(0)
        for c in gather_copies(0):
            c.wait()
        for c in out_copies(0, trow0):
            c.start()

        for c in out_copies(2, trow0 - T):
            c.wait()
        for c in out_copies(0, trow0):
            c.wait()

    return body(seq_blk, e1_blk, e2_blk, we, wpe)


def kernel(seq_inputs, e1_pos_inputs, e2_pos_inputs, we, wpe):
    b, l = seq_inputs.shape
    dw = we.shape[1]
    dp = wpe.shape[1]
    npos = wpe.shape[0]
    n = b * l
    assert n % (NW * T) == 0 and T % IB == 0
    assert (n // (NW * T)) % 3 == 1
    k = T // IB
    seq_blk = seq_inputs.reshape(n // T, k, IB)
    e1_blk = e1_pos_inputs.reshape(n // T, T)
    e2_blk = e2_pos_inputs.reshape(n // T, T)
    out = _sc_embed(seq_blk, e1_blk, e2_blk, we, wpe, n, dw, dp, npos)
    return out.reshape(b, l, dw + 2 * dp)
